# baseline (speedup 1.0000x reference)
import functools

import jax
import jax.numpy as jnp
from jax.experimental import pallas as pl
from jax.experimental.pallas import tpu as pltpu

_IN_CHUNK = 28
_OUT_CHUNK = 49


def _se_kernel(x_ref, w1t_ref, w2t_ref, o_ref, slab_ref, acc_ref, gate_ref,
               *, inv_hw, n_in):
    i = pl.program_id(0)

    @pl.when(i < n_in)
    def _pool():
        xt = x_ref[...]
        s = jnp.sum(xt, axis=0)

        @pl.when(i == 0)
        def _init():
            acc_ref[...] = s

        @pl.when(i > 0)
        def _accum():
            acc_ref[...] += s

        slab_ref[pl.ds(i * _IN_CHUNK, _IN_CHUNK)] = xt

    @pl.when(i >= n_in)
    def _scale():
        @pl.when(i == n_in)
        def _gates():
            y1 = jnp.dot(acc_ref[...] * inv_hw, w1t_ref[...],
                         preferred_element_type=jnp.float32)
            y1 = jnp.maximum(y1, 0.0)
            y2 = jnp.dot(y1, w2t_ref[...],
                         preferred_element_type=jnp.float32)
            gate_ref[...] = 1.0 / (1.0 + jnp.exp(-y2))

        j = i - n_in
        o_ref[...] = slab_ref[pl.ds(j * _OUT_CHUNK, _OUT_CHUNK)] * gate_ref[...]


def kernel(x, w1, w2):
    B, C, H, W = x.shape
    Cr = w1.shape[0]
    HW = H * W
    xv = x.transpose(2, 3, 0, 1).reshape(HW, B, C)
    w1t = w1.T
    w2t = w2.T

    n_in = HW // _IN_CHUNK
    n_out = HW // _OUT_CHUNK

    out = pl.pallas_call(
        functools.partial(_se_kernel, inv_hw=1.0 / float(HW), n_in=n_in),
        out_shape=jax.ShapeDtypeStruct((HW, B, C), x.dtype),
        grid=(n_in + n_out,),
        in_specs=[
            pl.BlockSpec((_IN_CHUNK, B, C),
                         lambda i: (jnp.minimum(i, 784 // _IN_CHUNK - 1), 0, 0)),
            pl.BlockSpec((C, Cr), lambda i: (0, 0)),
            pl.BlockSpec((Cr, C), lambda i: (0, 0)),
        ],
        out_specs=pl.BlockSpec(
            (_OUT_CHUNK, B, C),
            lambda i: (jnp.maximum(i - 784 // _IN_CHUNK, 0), 0, 0)),
        scratch_shapes=[
            pltpu.VMEM((HW, B, C), jnp.float32),
            pltpu.VMEM((B, C), jnp.float32),
            pltpu.VMEM((B, C), jnp.float32),
        ],
        compiler_params=pltpu.CompilerParams(
            dimension_semantics=("arbitrary",),
            vmem_limit_bytes=62 << 20),
    )(xv, w1t, w2t)
    return out.reshape(H, W, B, C).transpose(2, 3, 0, 1)
